# baseline (speedup 1.0000x reference)
import functools

import jax
import jax.numpy as jnp
from jax import lax
from jax.experimental import pallas as pl
from jax.experimental.pallas import tpu as pltpu
from jax.experimental.pallas import tpu_sc as plsc

VOCAB = 1000000
EMBED = 64
BATCH = 4096
HIST = 200

_NC = 2
_NS = 16
_NW = _NC * _NS
_BPW = BATCH // _NW
_CHUNK = 100
_NCHUNK = HIST // _CHUNK


def _sc_gather_pool(table, xr):
    mesh = plsc.VectorSubcoreMesh(core_axis_name="c", subcore_axis_name="s")

    @functools.partial(
        pl.kernel,
        out_type=jax.ShapeDtypeStruct((BATCH, EMBED), jnp.float32),
        mesh=mesh,
        scratch_types=[
            pltpu.VMEM((_BPW * _NCHUNK, _CHUNK), jnp.int32),
            pltpu.VMEM((2, HIST, EMBED), jnp.float32),
            pltpu.VMEM((_BPW, EMBED), jnp.float32),
            pltpu.SemaphoreType.DMA((2,)),
        ],
        compiler_params=pltpu.CompilerParams(use_tc_tiling_on_sc=False),
    )
    def k(table_hbm, x_hbm, out_hbm, idx_v, buf_v, pool_v, sems):
        wid = lax.axis_index("s") * _NC + lax.axis_index("c")
        row0 = wid * (_BPW * _NCHUNK)
        pltpu.sync_copy(x_hbm.at[pl.ds(row0, _BPW * _NCHUNK)], idx_v)

        def fire(i, slot):
            pltpu.async_copy(table_hbm.at[idx_v.at[_NCHUNK * i]],
                             buf_v.at[slot, pl.ds(0, _CHUNK)], sems.at[slot])
            pltpu.async_copy(table_hbm.at[idx_v.at[_NCHUNK * i + 1]],
                             buf_v.at[slot, pl.ds(_CHUNK, _CHUNK)],
                             sems.at[slot])

        fire(0, 0)

        def per_row(i, _):
            slot = lax.rem(i, 2)

            @pl.when(i + 1 < _BPW)
            def _():
                fire(i + 1, lax.rem(i + 1, 2))

            pltpu.make_async_copy(
                table_hbm.at[idx_v.at[0]],
                buf_v.at[slot, pl.ds(0, _CHUNK)], sems.at[slot]).wait()
            pltpu.make_async_copy(
                table_hbm.at[idx_v.at[0]],
                buf_v.at[slot, pl.ds(_CHUNK, _CHUNK)], sems.at[slot]).wait()

            def acc_body(r, acc):
                a0, a1, a2, a3 = acc
                a0 = a0 + buf_v[slot, r, pl.ds(0, 16)]
                a1 = a1 + buf_v[slot, r, pl.ds(16, 16)]
                a2 = a2 + buf_v[slot, r, pl.ds(32, 16)]
                a3 = a3 + buf_v[slot, r, pl.ds(48, 16)]
                return (a0, a1, a2, a3)

            z = jnp.zeros((16,), jnp.float32)
            acc0 = (z, z, z, z)
            a0, a1, a2, a3 = plsc.parallel_loop(
                0, HIST, 1, unroll=8, carry=acc0)(acc_body)
            s = jnp.float32(1.0 / HIST)
            pool_v[i, pl.ds(0, 16)] = a0 * s
            pool_v[i, pl.ds(16, 16)] = a1 * s
            pool_v[i, pl.ds(32, 16)] = a2 * s
            pool_v[i, pl.ds(48, 16)] = a3 * s
            return 0

        lax.fori_loop(0, _BPW, per_row, 0)
        pltpu.sync_copy(pool_v, out_hbm.at[pl.ds(wid * _BPW, _BPW)])

    return k(table, xr)


def _tc_finish_body(p_ref, wt_ref, b_ref, bng_ref, bnb_ref, lng_ref,
                    lnb_ref, o_ref):
    eps = 1e-5
    p = p_ref[...]
    h = jnp.dot(p, wt_ref[...], preferred_element_type=jnp.float32)
    h = h + b_ref[...]
    mu = jnp.mean(h, axis=0, keepdims=True)
    var = jnp.mean((h - mu) ** 2, axis=0, keepdims=True)
    hb = (h - mu) / jnp.sqrt(var + eps) * bng_ref[...] + bnb_ref[...]
    lmu = jnp.mean(hb, axis=1, keepdims=True)
    lvar = jnp.mean((hb - lmu) ** 2, axis=1, keepdims=True)
    o_ref[...] = (hb - lmu) / jnp.sqrt(lvar + eps) * lng_ref[...] + lnb_ref[...]


def _tc_finish(pooled, Wt, b, bn_gamma, bn_beta, ln_gamma, ln_beta):
    return pl.pallas_call(
        _tc_finish_body,
        out_shape=jax.ShapeDtypeStruct((BATCH, EMBED), jnp.float32),
    )(pooled, Wt, b, bn_gamma, bn_beta, ln_gamma, ln_beta)


def kernel(x, table, W, b, bn_gamma, bn_beta, ln_gamma, ln_beta):
    x = x.astype(jnp.int32)
    xr = x.reshape(BATCH * _NCHUNK, _CHUNK)
    pooled = _sc_gather_pool(table, xr)
    return _tc_finish(
        pooled, W.T, b.reshape(1, EMBED),
        bn_gamma.reshape(1, EMBED), bn_beta.reshape(1, EMBED),
        ln_gamma.reshape(1, EMBED), ln_beta.reshape(1, EMBED))


# speedup vs baseline: 1.0720x; 1.0720x over previous
import functools

import jax
import jax.numpy as jnp
from jax import lax
from jax.experimental import pallas as pl
from jax.experimental.pallas import tpu as pltpu
from jax.experimental.pallas import tpu_sc as plsc

VOCAB = 1000000
EMBED = 64
BATCH = 4096
HIST = 200

_NC = 2
_NS = 16
_NW = _NC * _NS
_BPW = BATCH // _NW


def _sc_gather_pool(table, xt3):
    mesh = plsc.VectorSubcoreMesh(core_axis_name="c", subcore_axis_name="s")

    @functools.partial(
        pl.kernel,
        out_type=jax.ShapeDtypeStruct((BATCH, EMBED), jnp.float32),
        mesh=mesh,
        scratch_types=[
            pltpu.VMEM((HIST, _BPW), jnp.int32),
            pltpu.VMEM((_BPW, EMBED), jnp.float32),
            pltpu.SemaphoreType.DMA,
        ],
        compiler_params=pltpu.CompilerParams(use_tc_tiling_on_sc=False),
    )
    def k(table_hbm, xt_hbm, out_hbm, xt_v, pool_v, sem):
        wid = lax.axis_index("s") * _NC + lax.axis_index("c")
        pltpu.sync_copy(xt_hbm.at[wid], xt_v)

        z = jnp.zeros((16,), jnp.float32)

        def zero_row(i, _):
            pool_v[i, pl.ds(0, 16)] = z
            pool_v[i, pl.ds(16, 16)] = z
            pool_v[i, pl.ds(32, 16)] = z
            pool_v[i, pl.ds(48, 16)] = z
            return 0

        lax.fori_loop(0, _BPW, zero_row, 0)

        def fire(j, _):
            pltpu.async_copy(table_hbm.at[xt_v.at[j]], pool_v, sem, add=True)
            return 0

        lax.fori_loop(0, HIST, fire, 0)

        def drain(j, _):
            pltpu.make_async_copy(table_hbm.at[xt_v.at[0]], pool_v, sem).wait()
            return 0

        lax.fori_loop(0, HIST, drain, 0)

        pltpu.sync_copy(pool_v, out_hbm.at[pl.ds(wid * _BPW, _BPW)])

    return k(table, xt3)


def _tc_finish_body(p_ref, wt_ref, b_ref, bng_ref, bnb_ref, lng_ref,
                    lnb_ref, o_ref):
    eps = 1e-5
    p = p_ref[...] * jnp.float32(1.0 / HIST)
    h = jnp.dot(p, wt_ref[...], preferred_element_type=jnp.float32)
    h = h + b_ref[...]
    mu = jnp.mean(h, axis=0, keepdims=True)
    var = jnp.mean((h - mu) ** 2, axis=0, keepdims=True)
    hb = (h - mu) / jnp.sqrt(var + eps) * bng_ref[...] + bnb_ref[...]
    lmu = jnp.mean(hb, axis=1, keepdims=True)
    lvar = jnp.mean((hb - lmu) ** 2, axis=1, keepdims=True)
    o_ref[...] = (hb - lmu) / jnp.sqrt(lvar + eps) * lng_ref[...] + lnb_ref[...]


def _tc_finish(pooled, Wt, b, bn_gamma, bn_beta, ln_gamma, ln_beta):
    return pl.pallas_call(
        _tc_finish_body,
        out_shape=jax.ShapeDtypeStruct((BATCH, EMBED), jnp.float32),
    )(pooled, Wt, b, bn_gamma, bn_beta, ln_gamma, ln_beta)


def kernel(x, table, W, b, bn_gamma, bn_beta, ln_gamma, ln_beta):
    x = x.astype(jnp.int32)
    xt3 = x.reshape(_NW, _BPW, HIST).transpose(0, 2, 1)
    pooled = _sc_gather_pool(table, xt3)
    return _tc_finish(
        pooled, W.T, b.reshape(1, EMBED),
        bn_gamma.reshape(1, EMBED), bn_beta.reshape(1, EMBED),
        ln_gamma.reshape(1, EMBED), ln_beta.reshape(1, EMBED))
